# Initial kernel scaffold; baseline (speedup 1.0000x reference)
import jax
import jax.numpy as jnp
from jax.experimental import pallas as pl


def kernel(hidden_states):
    raise NotImplementedError("write your pallas kernel here")


# speedup vs baseline: 2.4877x; 2.4877x over previous
import jax
import jax.numpy as jnp
from jax.experimental import pallas as pl

_INTERVAL_SIZE = 8
_I_FRAME_POSITION = 3
_THRESHOLD = 50.0
_PATCH_NUM = 196
_NUM_FRAMES = 32
_IMG_START = 64
_IMG_LEN = _PATCH_NUM * _NUM_FRAMES
_SEQ_LEN = 6400
_D_MODEL = 4096

_ROWS = 4
_NUM_BLOCKS = _SEQ_LEN // _ROWS
_PB_PER_FRAME = _PATCH_NUM // _ROWS
_IMG_BLOCK0 = _IMG_START // _ROWS
_NUM_IMG_BLOCKS = _IMG_LEN // _ROWS


def _body(x_ref, i_ref, o_ref):
    x = x_ref[...]
    i = i_ref[...]
    sad = jnp.sum(jnp.abs(x - i), axis=-1, keepdims=True)
    o_ref[...] = jnp.where(sad < _THRESHOLD, i, x)


def _i_index(g):
    b = g - _IMG_BLOCK0
    f = b // _PB_PER_FRAME
    p = b % _PB_PER_FRAME
    iframe = (f // _INTERVAL_SIZE) * _INTERVAL_SIZE + _I_FRAME_POSITION
    ig = _IMG_BLOCK0 + iframe * _PB_PER_FRAME + p
    in_img = jnp.logical_and(g >= _IMG_BLOCK0, g < _IMG_BLOCK0 + _NUM_IMG_BLOCKS)
    return jnp.where(in_img, ig, g)


def kernel(hidden_states):
    x = hidden_states.reshape(_NUM_BLOCKS, _ROWS, _D_MODEL)
    out = pl.pallas_call(
        _body,
        grid=(_NUM_BLOCKS,),
        in_specs=[
            pl.BlockSpec((1, _ROWS, _D_MODEL), lambda g: (g, 0, 0)),
            pl.BlockSpec((1, _ROWS, _D_MODEL), lambda g: (_i_index(g), 0, 0)),
        ],
        out_specs=pl.BlockSpec((1, _ROWS, _D_MODEL), lambda g: (g, 0, 0)),
        out_shape=jax.ShapeDtypeStruct((_NUM_BLOCKS, _ROWS, _D_MODEL), jnp.float32),
    )(x, x)
    return out.reshape(1, _SEQ_LEN, _D_MODEL)


# speedup vs baseline: 8.8688x; 3.5650x over previous
import jax
import jax.numpy as jnp
from jax.experimental import pallas as pl
from jax.experimental.pallas import tpu as pltpu

_INTERVAL_SIZE = 8
_I_FRAME_POSITION = 3
_THRESHOLD = 50.0
_PATCH_NUM = 196
_NUM_FRAMES = 32
_IMG_START = 64
_IMG_LEN = _PATCH_NUM * _NUM_FRAMES
_SEQ_LEN = 6400
_D_MODEL = 4096

_ROWS = 4
_NUM_GROUPS = _SEQ_LEN // _ROWS
_PB_PER_FRAME = _PATCH_NUM // _ROWS
_IMG_GROUP0 = _IMG_START // _ROWS
_NUM_IMG_GROUPS = _IMG_LEN // _ROWS
_PB_PER_INTERVAL = _PB_PER_FRAME * _INTERVAL_SIZE
_IF_SLOT_PITCH = 50
_NUM_INTERVALS = 4

_B = 32
_NSTEPS = _NUM_GROUPS // _B


def _iframe_copies(x_any, if_ref, sem):
    copies = []
    for i in range(_NUM_INTERVALS):
        src = _IMG_GROUP0 + _PB_PER_FRAME * (i * _INTERVAL_SIZE + _I_FRAME_POSITION)
        copies.append(
            pltpu.make_async_copy(
                x_any.at[pl.ds(src, _PB_PER_FRAME)],
                if_ref.at[pl.ds(i * _IF_SLOT_PITCH, _PB_PER_FRAME)],
                sem,
            )
        )
    return copies


def _body(x_ref, x_any, o_ref, if_ref, sem):
    g = pl.program_id(0)

    @pl.when(g == 0)
    def _load_iframes():
        copies = _iframe_copies(x_any, if_ref, sem)
        for c in copies:
            c.start()
        for c in copies:
            c.wait()

    for j in range(_B):
        gg = g * _B + j
        in_img = jnp.logical_and(gg >= _IMG_GROUP0, gg < _IMG_GROUP0 + _NUM_IMG_GROUPS)
        b = gg - _IMG_GROUP0
        slot = jnp.where(
            in_img, (b // _PB_PER_INTERVAL) * _IF_SLOT_PITCH + b % _PB_PER_FRAME, 0
        )
        x_grp = x_ref[j]
        i_grp = if_ref[slot]
        i_eff = jnp.where(in_img, i_grp, x_grp)
        sad = jnp.sum(jnp.abs(x_grp - i_eff), axis=-1, keepdims=True)
        o_ref[j] = jnp.where(sad < _THRESHOLD, i_eff, x_grp)


def kernel(hidden_states):
    x = hidden_states.reshape(_NUM_GROUPS, _ROWS, _D_MODEL)
    out = pl.pallas_call(
        _body,
        grid=(_NSTEPS,),
        in_specs=[
            pl.BlockSpec((_B, _ROWS, _D_MODEL), lambda g: (g, 0, 0)),
            pl.BlockSpec(memory_space=pl.ANY),
        ],
        out_specs=pl.BlockSpec((_B, _ROWS, _D_MODEL), lambda g: (g, 0, 0)),
        out_shape=jax.ShapeDtypeStruct((_NUM_GROUPS, _ROWS, _D_MODEL), jnp.float32),
        scratch_shapes=[
            pltpu.VMEM((_NUM_INTERVALS * _IF_SLOT_PITCH, _ROWS, _D_MODEL), jnp.float32),
            pltpu.SemaphoreType.DMA,
        ],
    )(x, x)
    return out.reshape(1, _SEQ_LEN, _D_MODEL)


# speedup vs baseline: 27.7966x; 3.1342x over previous
import jax
import jax.numpy as jnp
from jax.experimental import pallas as pl
from jax.experimental.pallas import tpu as pltpu

_INTERVAL_SIZE = 8
_I_FRAME_POSITION = 3
_THRESHOLD = 50.0
_PATCH_NUM = 196
_NUM_FRAMES = 32
_IMG_START = 64
_IMG_LEN = _PATCH_NUM * _NUM_FRAMES
_SEQ_LEN = 6400
_D_MODEL = 4096
_NUM_INTERVALS = 4

_ROWS = 4
_IMG_GROUP0 = _IMG_START // _ROWS
_NUM_IMG_GROUPS = _IMG_LEN // _ROWS
_PB_PER_FRAME = _PATCH_NUM // _ROWS
_PB_PER_INTERVAL = _PB_PER_FRAME * _INTERVAL_SIZE
_STAGE_PITCH = 200

_BLOCK_ROWS = 128
_GROUPS_PER_STEP = _BLOCK_ROWS // _ROWS
_NSTEPS = _SEQ_LEN // _BLOCK_ROWS


def _body(x_ref, x_any, o_ref, stage_ref, if_ref, sem):
    g = pl.program_id(0)

    @pl.when(g == 0)
    def _load_iframes():
        copies = []
        for i in range(_NUM_INTERVALS):
            src = _IMG_START + _PATCH_NUM * (i * _INTERVAL_SIZE + _I_FRAME_POSITION) - 4
            copies.append(
                pltpu.make_async_copy(
                    x_any.at[0, pl.ds(src, _STAGE_PITCH)],
                    stage_ref.at[pl.ds(i * _STAGE_PITCH, _STAGE_PITCH)],
                    sem,
                )
            )
        for c in copies:
            c.start()
        for c in copies:
            c.wait()
        for i in range(_NUM_INTERVALS):
            for p in range(_PB_PER_FRAME):
                lo = i * _STAGE_PITCH + 4 + _ROWS * p
                if_ref[i * _PB_PER_FRAME + p] = stage_ref[lo : lo + _ROWS, :]

    for j in range(_GROUPS_PER_STEP):
        gg = g * _GROUPS_PER_STEP + j
        in_img = jnp.logical_and(gg >= _IMG_GROUP0, gg < _IMG_GROUP0 + _NUM_IMG_GROUPS)
        b = gg - _IMG_GROUP0
        slot = jnp.where(
            in_img, (b // _PB_PER_INTERVAL) * _PB_PER_FRAME + b % _PB_PER_FRAME, 0
        )
        x_grp = x_ref[0, _ROWS * j : _ROWS * (j + 1), :]
        i_grp = if_ref[slot]
        i_eff = jnp.where(in_img, i_grp, x_grp)
        sad = jnp.sum(jnp.abs(x_grp - i_eff), axis=-1, keepdims=True)
        o_ref[0, _ROWS * j : _ROWS * (j + 1), :] = jnp.where(
            sad < _THRESHOLD, i_eff, x_grp
        )


def kernel(hidden_states):
    return pl.pallas_call(
        _body,
        grid=(_NSTEPS,),
        in_specs=[
            pl.BlockSpec((1, _BLOCK_ROWS, _D_MODEL), lambda g: (0, g, 0)),
            pl.BlockSpec(memory_space=pl.ANY),
        ],
        out_specs=pl.BlockSpec((1, _BLOCK_ROWS, _D_MODEL), lambda g: (0, g, 0)),
        out_shape=jax.ShapeDtypeStruct((1, _SEQ_LEN, _D_MODEL), jnp.float32),
        scratch_shapes=[
            pltpu.VMEM((_NUM_INTERVALS * _STAGE_PITCH, _D_MODEL), jnp.float32),
            pltpu.VMEM(
                (_NUM_INTERVALS * _PB_PER_FRAME, _ROWS, _D_MODEL), jnp.float32
            ),
            pltpu.SemaphoreType.DMA,
        ],
    )(hidden_states, hidden_states)


# speedup vs baseline: 31.2680x; 1.1249x over previous
import jax
import jax.numpy as jnp
from jax.experimental import pallas as pl
from jax.experimental.pallas import tpu as pltpu

_INTERVAL_SIZE = 8
_I_FRAME_POSITION = 3
_THRESHOLD = 50.0
_PATCH_NUM = 196
_NUM_FRAMES = 32
_IMG_START = 64
_IMG_LEN = _PATCH_NUM * _NUM_FRAMES
_SEQ_LEN = 6400
_D_MODEL = 4096
_NUM_INTERVALS = 4

_ROWS = 4
_IMG_GROUP0 = _IMG_START // _ROWS
_NUM_IMG_GROUPS = _IMG_LEN // _ROWS
_PB_PER_FRAME = _PATCH_NUM // _ROWS
_PB_PER_INTERVAL = _PB_PER_FRAME * _INTERVAL_SIZE
_STAGE_PITCH = 200

_BLOCK_ROWS = 256
_GROUPS_PER_STEP = _BLOCK_ROWS // _ROWS
_NSTEPS = _SEQ_LEN // _BLOCK_ROWS


def _stage_copy(x_any, stage_ref, sems, i):
    src = _IMG_START + _PATCH_NUM * (i * _INTERVAL_SIZE + _I_FRAME_POSITION) - 4
    return pltpu.make_async_copy(
        x_any.at[0, pl.ds(src, _STAGE_PITCH)],
        stage_ref.at[pl.ds(i * _STAGE_PITCH, _STAGE_PITCH)],
        sems.at[i],
    )


def _repack_interval(stage_ref, if_ref, i):
    for p in range(_PB_PER_FRAME):
        lo = i * _STAGE_PITCH + 4 + _ROWS * p
        if_ref[i * _PB_PER_FRAME + p] = stage_ref[lo : lo + _ROWS, :]


def _body(x_ref, x_any, o_ref, stage_ref, if_ref, sems):
    g = pl.program_id(0)

    @pl.when(g == 0)
    def _start_loads():
        for i in range(_NUM_INTERVALS):
            _stage_copy(x_any, stage_ref, sems, i).start()

    for i in range(_NUM_INTERVALS):
        first_use = (_IMG_GROUP0 + _PB_PER_INTERVAL * i) // _GROUPS_PER_STEP
        repack_step = max(0, first_use - 1)

        @pl.when(g == repack_step)
        def _do_repack(i=i):
            _stage_copy(x_any, stage_ref, sems, i).wait()
            _repack_interval(stage_ref, if_ref, i)

    for j in range(_GROUPS_PER_STEP):
        gg = g * _GROUPS_PER_STEP + j
        in_img = jnp.logical_and(gg >= _IMG_GROUP0, gg < _IMG_GROUP0 + _NUM_IMG_GROUPS)
        b = gg - _IMG_GROUP0
        slot = jnp.where(
            in_img, (b // _PB_PER_INTERVAL) * _PB_PER_FRAME + b % _PB_PER_FRAME, 0
        )
        x_grp = x_ref[0, _ROWS * j : _ROWS * (j + 1), :]
        i_grp = if_ref[slot]
        i_eff = jnp.where(in_img, i_grp, x_grp)
        sad = jnp.sum(jnp.abs(x_grp - i_eff), axis=-1, keepdims=True)
        o_ref[0, _ROWS * j : _ROWS * (j + 1), :] = jnp.where(
            sad < _THRESHOLD, i_eff, x_grp
        )


def kernel(hidden_states):
    return pl.pallas_call(
        _body,
        grid=(_NSTEPS,),
        in_specs=[
            pl.BlockSpec((1, _BLOCK_ROWS, _D_MODEL), lambda g: (0, g, 0)),
            pl.BlockSpec(memory_space=pl.ANY),
        ],
        out_specs=pl.BlockSpec((1, _BLOCK_ROWS, _D_MODEL), lambda g: (0, g, 0)),
        out_shape=jax.ShapeDtypeStruct((1, _SEQ_LEN, _D_MODEL), jnp.float32),
        scratch_shapes=[
            pltpu.VMEM((_NUM_INTERVALS * _STAGE_PITCH, _D_MODEL), jnp.float32),
            pltpu.VMEM(
                (_NUM_INTERVALS * _PB_PER_FRAME, _ROWS, _D_MODEL), jnp.float32
            ),
            pltpu.SemaphoreType.DMA((_NUM_INTERVALS,)),
        ],
        compiler_params=pltpu.CompilerParams(
            vmem_limit_bytes=100 * 1024 * 1024,
        ),
    )(hidden_states, hidden_states)


# speedup vs baseline: 31.8885x; 1.0198x over previous
import jax
import jax.numpy as jnp
from jax.experimental import pallas as pl
from jax.experimental.pallas import tpu as pltpu

_INTERVAL_SIZE = 8
_I_FRAME_POSITION = 3
_THRESHOLD = 50.0
_PATCH_NUM = 196
_NUM_FRAMES = 32
_IMG_START = 64
_IMG_LEN = _PATCH_NUM * _NUM_FRAMES
_SEQ_LEN = 6400
_D_MODEL = 4096
_NUM_INTERVALS = 4

_ROWS = 4
_IMG_GROUP0 = _IMG_START // _ROWS
_NUM_IMG_GROUPS = _IMG_LEN // _ROWS
_PB_PER_FRAME = _PATCH_NUM // _ROWS
_PB_PER_INTERVAL = _PB_PER_FRAME * _INTERVAL_SIZE
_STAGE_PITCH = 200

_BLOCK_ROWS = 320
_GROUPS_PER_STEP = _BLOCK_ROWS // _ROWS
_NSTEPS = _SEQ_LEN // _BLOCK_ROWS


def _stage_copy(x_any, stage_ref, sems, i):
    src = _IMG_START + _PATCH_NUM * (i * _INTERVAL_SIZE + _I_FRAME_POSITION) - 4
    return pltpu.make_async_copy(
        x_any.at[0, pl.ds(src, _STAGE_PITCH)],
        stage_ref.at[pl.ds(i * _STAGE_PITCH, _STAGE_PITCH)],
        sems.at[i],
    )


def _repack_interval(stage_ref, if_ref, i):
    for p in range(_PB_PER_FRAME):
        lo = i * _STAGE_PITCH + 4 + _ROWS * p
        if_ref[i * _PB_PER_FRAME + p] = stage_ref[lo : lo + _ROWS, :]


def _body(x_ref, x_any, o_ref, stage_ref, if_ref, sems):
    g = pl.program_id(0)

    @pl.when(g == 0)
    def _start_loads():
        for i in range(_NUM_INTERVALS):
            _stage_copy(x_any, stage_ref, sems, i).start()

    for i in range(_NUM_INTERVALS):
        first_use = (_IMG_GROUP0 + _PB_PER_INTERVAL * i) // _GROUPS_PER_STEP
        repack_step = max(0, first_use - 1)

        @pl.when(g == repack_step)
        def _do_repack(i=i):
            _stage_copy(x_any, stage_ref, sems, i).wait()
            _repack_interval(stage_ref, if_ref, i)

    for j in range(_GROUPS_PER_STEP):
        gg = g * _GROUPS_PER_STEP + j
        in_img = jnp.logical_and(gg >= _IMG_GROUP0, gg < _IMG_GROUP0 + _NUM_IMG_GROUPS)
        b = gg - _IMG_GROUP0
        slot = jnp.where(
            in_img, (b // _PB_PER_INTERVAL) * _PB_PER_FRAME + b % _PB_PER_FRAME, 0
        )
        x_grp = x_ref[0, _ROWS * j : _ROWS * (j + 1), :]
        i_grp = if_ref[slot]
        i_eff = jnp.where(in_img, i_grp, x_grp)
        sad = jnp.sum(jnp.abs(x_grp - i_eff), axis=-1, keepdims=True)
        o_ref[0, _ROWS * j : _ROWS * (j + 1), :] = jnp.where(
            sad < _THRESHOLD, i_eff, x_grp
        )


def kernel(hidden_states):
    return pl.pallas_call(
        _body,
        grid=(_NSTEPS,),
        in_specs=[
            pl.BlockSpec((1, _BLOCK_ROWS, _D_MODEL), lambda g: (0, g, 0)),
            pl.BlockSpec(memory_space=pl.ANY),
        ],
        out_specs=pl.BlockSpec((1, _BLOCK_ROWS, _D_MODEL), lambda g: (0, g, 0)),
        out_shape=jax.ShapeDtypeStruct((1, _SEQ_LEN, _D_MODEL), jnp.float32),
        scratch_shapes=[
            pltpu.VMEM((_NUM_INTERVALS * _STAGE_PITCH, _D_MODEL), jnp.float32),
            pltpu.VMEM(
                (_NUM_INTERVALS * _PB_PER_FRAME, _ROWS, _D_MODEL), jnp.float32
            ),
            pltpu.SemaphoreType.DMA((_NUM_INTERVALS,)),
        ],
        compiler_params=pltpu.CompilerParams(
            vmem_limit_bytes=100 * 1024 * 1024,
        ),
    )(hidden_states, hidden_states)
